# Initial kernel scaffold; baseline (speedup 1.0000x reference)
import jax
import jax.numpy as jnp
from jax.experimental import pallas as pl


def kernel(query, sequence, sequence_length, W1, b1, W2, b2, Wl, bl, alpha):
    raise NotImplementedError("write your pallas kernel here")


# speedup vs baseline: 1.0655x; 1.0655x over previous
import jax
import jax.numpy as jnp
from jax.experimental import pallas as pl
from jax.experimental.pallas import tpu as pltpu

B = 4096
L = 200
D = 32
H1 = 64
H2 = 32
WINDOWS = (10, 20, 50, 120)
CUMSUM = (0, 10, 30, 80)
BB = 128


def _din_block(q_ref, seq_ref, len_ref, w1ab_ref, w1c_ref, b1_ref, w2_ref,
               b2_ref, wl_ref, bl_ref, alpha_ref, out_ref):
    q = q_ref[:]
    seq = seq_ref[:]
    qs = seq * q[:, None, :]
    x = jnp.concatenate([seq, qs], axis=-1).reshape(BB * L, 2 * D)
    h = jnp.dot(x, w1ab_ref[:], preferred_element_type=jnp.float32)
    qc = jnp.dot(q, w1c_ref[:], preferred_element_type=jnp.float32) + b1_ref[:]
    h = jnp.maximum(h.reshape(BB, L, H1) + qc[:, None, :], 0.0)
    h2 = jnp.dot(h.reshape(BB * L, H1), w2_ref[:],
                 preferred_element_type=jnp.float32) + b2_ref[:]
    h2 = jnp.maximum(h2, 0.0)
    a = jnp.sum(h2 * wl_ref[:], axis=-1, keepdims=True) + bl_ref[0, 0]
    alpha = alpha_ref[0, 0]
    a = jnp.where(a > 0, a, alpha * a).reshape(BB, L)
    slen = len_ref[:]
    tpos = jax.lax.broadcasted_iota(jnp.int32, (BB, L), 1)
    w = jnp.where(tpos < slen, a, 0.0)
    parts = []
    for st, wn in zip(CUMSUM, WINDOWS):
        sw = jnp.sum(seq[:, st:st + wn, :] * w[:, st:st + wn, None], axis=1)
        denom = jnp.maximum(jnp.minimum(slen - st, wn), 1).astype(jnp.float32)
        parts.append(sw / denom)
    parts.append(q)
    out_ref[:] = jnp.concatenate(parts, axis=-1)


def kernel(query, sequence, sequence_length, W1, b1, W2, b2, Wl, bl, alpha):
    slen = sequence_length.astype(jnp.int32).reshape(B, 1)
    w1ab = W1[:2 * D]
    w1c = W1[2 * D:]
    b1r = b1.reshape(1, H1)
    b2r = b2.reshape(1, H2)
    wlr = Wl.reshape(1, H2)
    blr = bl.reshape(1, 1)
    alphar = alpha.reshape(1, 1)

    grid = (B // BB,)
    full = lambda *s: pl.BlockSpec(s, lambda i: (0,) * len(s))
    out = pl.pallas_call(
        _din_block,
        grid=grid,
        in_specs=[
            pl.BlockSpec((BB, D), lambda i: (i, 0)),
            pl.BlockSpec((BB, L, D), lambda i: (i, 0, 0)),
            pl.BlockSpec((BB, 1), lambda i: (i, 0)),
            full(2 * D, H1),
            full(D, H1),
            full(1, H1),
            full(H1, H2),
            full(1, H2),
            full(1, H2),
            full(1, 1),
            full(1, 1),
        ],
        out_specs=pl.BlockSpec((BB, 5 * D), lambda i: (i, 0)),
        out_shape=jax.ShapeDtypeStruct((B, 5 * D), jnp.float32),
        compiler_params=pltpu.CompilerParams(
            dimension_semantics=("arbitrary",),
        ),
    )(query, sequence, slen, w1ab, w1c, b1r, W2, b2r, wlr, blr, alphar)
    return out
